# baseline (speedup 1.0000x reference)
import jax
import jax.numpy as jnp
from jax import lax
from jax.experimental import pallas as pl
from jax.experimental.pallas import tpu as pltpu
from jax.experimental.pallas import tpu_sc as plsc

_B = 8
_NQ = 8
_L = 2048
_D = 256
_NW = 32
_LPW = _L // (_NW // _B)
_LC = 8
_NCH = _LPW // _LC
_NB = 4
_K = 2


def _sc_body(x_hbm, emb_hbm, out_hbm, emb_v, bufs, ld_sems, st_sems):
    wid = lax.axis_index("s") * 2 + lax.axis_index("c")
    b = wid // 4
    lbase = (wid - b * 4) * _LPW
    pltpu.sync_copy(emb_hbm, emb_v)

    e = [
        [emb_v[qi, pl.ds(j * 16, 16)] for j in range(16)]
        for qi in range(_NQ)
    ]

    def start_load(cc, p):
        l0 = lbase + cc * _LC
        for qi in range(_NQ):
            pltpu.async_copy(
                x_hbm.at[b, qi, pl.ds(l0, _LC), :],
                bufs.at[p, :, pl.ds(qi * _D, _D)],
                ld_sems.at[p],
            )

    def wait_load(p):
        pltpu.make_async_copy(
            out_hbm.at[0, pl.ds(0, _LC), :], bufs.at[p], ld_sems.at[p]
        ).wait()

    def start_store(cc, p):
        l0 = lbase + cc * _LC
        pltpu.async_copy(
            bufs.at[p], out_hbm.at[b, pl.ds(l0, _LC), :], st_sems.at[p]
        )

    def wait_store(p):
        pltpu.make_async_copy(
            bufs.at[p], out_hbm.at[0, pl.ds(0, _LC), :], st_sems.at[p]
        ).wait()

    for cc in range(_K):
        start_load(cc, cc % _NB)

    def round_body(r, carry):
        for par in range(_NB):
            cc = r * _NB + par
            wait_load(par)
            pn = (par + _K) % _NB
            nxt = cc + _K

            @pl.when(nxt >= _NB)
            def _():
                wait_store(pn)

            @pl.when(nxt < _NCH)
            def _():
                start_load(nxt, pn)

            for qi in range(_NQ):
                @plsc.parallel_loop(0, _LC, unroll=2)
                def row(l, _p=par, _qi=qi):
                    for j in range(16):
                        sl = pl.ds(_qi * _D + j * 16, 16)
                        bufs[_p, l, sl] = bufs[_p, l, sl] + e[_qi][j]

            start_store(cc, par)
        return carry

    lax.fori_loop(0, _NCH // _NB, round_body, 0)
    for cc in range(_NCH - _K, _NCH):
        wait_store(cc % _NB)


@jax.jit
def _sc_call(x, quantizer_emb):
    mesh = plsc.VectorSubcoreMesh(core_axis_name="c", subcore_axis_name="s")
    f = pl.kernel(
        _sc_body,
        out_type=jax.ShapeDtypeStruct((_B, _L, _NQ * _D), jnp.float32),
        mesh=mesh,
        scratch_types=[
            pltpu.VMEM((_NQ, _D), jnp.float32),
            pltpu.VMEM((_NB, _LC, _NQ * _D), jnp.float32),
            pltpu.SemaphoreType.DMA((_NB,)),
            pltpu.SemaphoreType.DMA((_NB,)),
        ],
    )
    return f(x, quantizer_emb)


def kernel(x, quantizer_emb):
    return _sc_call(x, quantizer_emb)
